# baseline (speedup 1.0000x reference)
import functools

import jax
import jax.numpy as jnp
from jax import lax
from jax.experimental import pallas as pl
from jax.experimental.pallas import tpu as pltpu
from jax.experimental.pallas import tpu_sc as plsc

_N = 1048576
_D = 64
_PD = 128
_ND = 101
_NT = 22
_NP = 8
_TBL = _ND * _NT * _NP

_NW = 32
_NSLAB = 8
_NS = _N // _NSLAB
_RPW = _NS // _NW
_C = 256
_NCHUNK = _RPW // _C


def _table_body(pe_ref, emb_ref, wp_ref, bp_ref, wo_ref, bo_ref, g_ref, b_ref,
                out_ref):
    wo = wo_ref[:]
    a = jnp.dot(pe_ref[:], wo[0:32, :], preferred_element_type=jnp.float32)
    b = jnp.dot(emb_ref[:], wo[32:48, :], preferred_element_type=jnp.float32)
    wp_o = jnp.dot(wp_ref[:], wo[48:64, :], preferred_element_type=jnp.float32)
    base = (jnp.dot(bp_ref[:], wo[48:64, :], preferred_element_type=jnp.float32)
            + bo_ref[:])
    pvals = lax.broadcasted_iota(jnp.int32, (_NP, 1), 0).astype(jnp.float32)
    c = pvals * wp_o + base
    x = (a[:, None, None, :] + b[None, :, None, :] + c[None, None, :, :])
    mean = jnp.sum(x, axis=-1, keepdims=True) * (1.0 / _D)
    var = jnp.sum(x * x, axis=-1, keepdims=True) * (1.0 / _D) - mean * mean
    out_ref[:] = (x - mean) * lax.rsqrt(var + 1e-5) * g_ref[:] + b_ref[:]


def _build_table(depth_pe, embed_table, Wp, bp, Wo, bo, gamma, beta):
    pad = _PD - _D
    table4 = pl.pallas_call(
        _table_body,
        out_shape=jax.ShapeDtypeStruct((_ND, _NT, _NP, _PD), jnp.float32),
    )(depth_pe, embed_table, Wp, bp.reshape(1, -1),
      jnp.pad(Wo, ((0, 0), (0, pad))), jnp.pad(bo, (0, pad)).reshape(1, -1),
      jnp.pad(gamma, (0, pad)).reshape(1, -1),
      jnp.pad(beta, (0, pad)).reshape(1, -1))
    return table4.reshape(_TBL, _PD)


def _gather_body(d_hbm, t_hbm, p_hbm, table_hbm, out_hbm,
                 d_v, t_v, p_v, idx0_v, idx1_v, rows0_v, rows1_v, packt_v,
                 sem):
    idx_s = (idx0_v, idx1_v)
    rows_s = (rows0_v, rows1_v)
    wid = lax.axis_index("s") * 2 + lax.axis_index("c")
    base = wid * _RPW
    last = _NCHUNK - 1

    def load_idx(slot, ci):
        off = base + ci * _C
        pltpu.sync_copy(d_hbm.at[pl.ds(off, _C)], d_v)
        pltpu.sync_copy(t_hbm.at[pl.ds(off, _C)], t_v)
        pltpu.sync_copy(p_hbm.at[pl.ds(off, _C)], p_v)
        for i in range(_C // 16):
            s = pl.ds(i * 16, 16)
            d = jnp.clip(d_v[s], 0, _ND - 1)
            t = jnp.clip(t_v[s], 0, _NT - 1)
            idx_s[slot][s] = d * (_NT * _NP) + t * _NP + p_v[s]

    def start_gather(slot):
        return pltpu.async_copy(table_hbm.at[idx_s[slot]],
                                rows_s[slot], sem)

    def wait_gather(slot):
        pltpu.make_async_copy(table_hbm.at[idx_s[slot]],
                              rows_s[slot], sem).wait()

    load_idx(0, 0)
    start_gather(0)
    load_idx(1, 1)

    def outer(gi, carry):
        for b in (0, 1):
            g = 2 * gi + b
            nb = 1 - b
            wait_gather(b)
            start_gather(nb)

            def pack_row(r, carry, _rows=rows_s[b]):
                for k in range(_D // 16):
                    s = pl.ds(k * 16, 16)
                    packt_v[r, s] = _rows[r, s]
                return carry

            lax.fori_loop(0, _C, pack_row, 0)
            pltpu.sync_copy(packt_v, out_hbm.at[pl.ds(base + g * _C, _C)])
            load_idx(b, jnp.minimum(g + 2, last))
        return carry

    lax.fori_loop(0, _NCHUNK // 2, outer, 0)
    wait_gather(0)


@functools.cache
def _make_gather():
    return functools.partial(
        pl.kernel,
        out_type=jax.ShapeDtypeStruct((_NS, _D), jnp.float32),
        mesh=plsc.VectorSubcoreMesh(core_axis_name="c", subcore_axis_name="s",
                                    num_cores=2, num_subcores=16),
        scratch_types=[
            pltpu.VMEM((_C,), jnp.int32),
            pltpu.VMEM((_C,), jnp.int32),
            pltpu.VMEM((_C,), jnp.int32),
            pltpu.VMEM((_C,), jnp.int32),
            pltpu.VMEM((_C,), jnp.int32),
            pltpu.VMEM((_C, _PD), jnp.float32),
            pltpu.VMEM((_C, _PD), jnp.float32),
            pltpu.VMEM((_C, _D), jnp.float32),
            pltpu.SemaphoreType.DMA,
        ],
        compiler_params=pltpu.CompilerParams(use_tc_tiling_on_sc=True),
    )(_gather_body)


_TB = 8192
_NB = _NS // _TB


def _tpose_first_body(x_ref, o_ref):
    o_ref[:] = x_ref[:].T


def _tpose_acc_body(acc_ref, x_ref, o_ref):
    del acc_ref
    o_ref[:] = x_ref[:].T


def _transpose_slab(acc, slab, k):
    out_sd = jax.ShapeDtypeStruct((_D, _N), jnp.float32)
    out_spec = pl.BlockSpec((_D, _TB), lambda i, _k=k: (0, _k * _NB + i))
    slab_spec = pl.BlockSpec((_TB, _D), lambda i: (i, 0))
    if acc is None:
        return pl.pallas_call(
            _tpose_first_body,
            grid=(_NB,),
            in_specs=[slab_spec],
            out_specs=out_spec,
            out_shape=out_sd,
        )(slab)
    return pl.pallas_call(
        _tpose_acc_body,
        grid=(_NB,),
        in_specs=[pl.BlockSpec(memory_space=pl.ANY), slab_spec],
        out_specs=out_spec,
        out_shape=out_sd,
        input_output_aliases={0: 0},
    )(acc, slab)


def kernel(derivation_depths, inference_types, parent_counts, depth_pe,
           embed_table, Wp, bp, Wo, bo, gamma, beta):
    table = _build_table(depth_pe, embed_table, Wp, bp, Wo, bo, gamma, beta)
    gather = _make_gather()
    acc = None
    for k in range(_NSLAB):
        s = slice(k * _NS, (k + 1) * _NS)
        slab = gather(derivation_depths[s], inference_types[s],
                      parent_counts[s], table)
        acc = _transpose_slab(acc, slab, k)
    return acc.T


# speedup vs baseline: 1.0137x; 1.0137x over previous
import functools

import jax
import jax.numpy as jnp
from jax import lax
from jax.experimental import pallas as pl
from jax.experimental.pallas import tpu as pltpu
from jax.experimental.pallas import tpu_sc as plsc

_N = 1048576
_D = 64
_PD = 128
_ND = 101
_NT = 22
_NP = 8
_TBL = _ND * _NT * _NP

_NW = 32
_NSLAB = 4
_NS = _N // _NSLAB
_RPW = _NS // _NW
_C = 256
_NCHUNK = _RPW // _C


def _table_body(pe_ref, emb_ref, wp_ref, bp_ref, wo_ref, bo_ref, g_ref, b_ref,
                out_ref):
    wo = wo_ref[:]
    a = jnp.dot(pe_ref[:], wo[0:32, :], preferred_element_type=jnp.float32)
    b = jnp.dot(emb_ref[:], wo[32:48, :], preferred_element_type=jnp.float32)
    wp_o = jnp.dot(wp_ref[:], wo[48:64, :], preferred_element_type=jnp.float32)
    base = (jnp.dot(bp_ref[:], wo[48:64, :], preferred_element_type=jnp.float32)
            + bo_ref[:])
    pvals = lax.broadcasted_iota(jnp.int32, (_NP, 1), 0).astype(jnp.float32)
    c = pvals * wp_o + base
    x = (a[:, None, None, :] + b[None, :, None, :] + c[None, None, :, :])
    mean = jnp.sum(x, axis=-1, keepdims=True) * (1.0 / _D)
    var = jnp.sum(x * x, axis=-1, keepdims=True) * (1.0 / _D) - mean * mean
    out_ref[:] = (x - mean) * lax.rsqrt(var + 1e-5) * g_ref[:] + b_ref[:]


def _build_table(depth_pe, embed_table, Wp, bp, Wo, bo, gamma, beta):
    pad = _PD - _D
    table4 = pl.pallas_call(
        _table_body,
        out_shape=jax.ShapeDtypeStruct((_ND, _NT, _NP, _PD), jnp.float32),
    )(depth_pe, embed_table, Wp, bp.reshape(1, -1),
      jnp.pad(Wo, ((0, 0), (0, pad))), jnp.pad(bo, (0, pad)).reshape(1, -1),
      jnp.pad(gamma, (0, pad)).reshape(1, -1),
      jnp.pad(beta, (0, pad)).reshape(1, -1))
    return table4.reshape(_TBL, _PD)


def _gather_body(d_hbm, t_hbm, p_hbm, table_hbm, out_hbm,
                 d_v, t_v, p_v, idx0_v, idx1_v, rows0_v, rows1_v, packt_v,
                 sem):
    idx_s = (idx0_v, idx1_v)
    rows_s = (rows0_v, rows1_v)
    wid = lax.axis_index("s") * 2 + lax.axis_index("c")
    base = wid * _RPW
    last = _NCHUNK - 1

    def load_idx(slot, ci):
        off = base + ci * _C
        pltpu.sync_copy(d_hbm.at[pl.ds(off, _C)], d_v)
        pltpu.sync_copy(t_hbm.at[pl.ds(off, _C)], t_v)
        pltpu.sync_copy(p_hbm.at[pl.ds(off, _C)], p_v)
        for i in range(_C // 16):
            s = pl.ds(i * 16, 16)
            d = jnp.clip(d_v[s], 0, _ND - 1)
            t = jnp.clip(t_v[s], 0, _NT - 1)
            idx_s[slot][s] = d * (_NT * _NP) + t * _NP + p_v[s]

    def start_gather(slot):
        return pltpu.async_copy(table_hbm.at[idx_s[slot]],
                                rows_s[slot], sem)

    def wait_gather(slot):
        pltpu.make_async_copy(table_hbm.at[idx_s[slot]],
                              rows_s[slot], sem).wait()

    load_idx(0, 0)
    start_gather(0)
    load_idx(1, 1)

    def outer(gi, carry):
        for b in (0, 1):
            g = 2 * gi + b
            nb = 1 - b
            wait_gather(b)
            start_gather(nb)

            def pack_row(r, carry, _rows=rows_s[b]):
                for k in range(_D // 16):
                    s = pl.ds(k * 16, 16)
                    packt_v[r, s] = _rows[r, s]
                return carry

            lax.fori_loop(0, _C, pack_row, 0)
            pltpu.sync_copy(packt_v, out_hbm.at[pl.ds(base + g * _C, _C)])
            load_idx(b, jnp.minimum(g + 2, last))
        return carry

    lax.fori_loop(0, _NCHUNK // 2, outer, 0)
    wait_gather(0)


@functools.cache
def _make_gather():
    return functools.partial(
        pl.kernel,
        out_type=jax.ShapeDtypeStruct((_NS, _D), jnp.float32),
        mesh=plsc.VectorSubcoreMesh(core_axis_name="c", subcore_axis_name="s",
                                    num_cores=2, num_subcores=16),
        scratch_types=[
            pltpu.VMEM((_C,), jnp.int32),
            pltpu.VMEM((_C,), jnp.int32),
            pltpu.VMEM((_C,), jnp.int32),
            pltpu.VMEM((_C,), jnp.int32),
            pltpu.VMEM((_C,), jnp.int32),
            pltpu.VMEM((_C, _PD), jnp.float32),
            pltpu.VMEM((_C, _PD), jnp.float32),
            pltpu.VMEM((_C, _D), jnp.float32),
            pltpu.SemaphoreType.DMA,
        ],
        compiler_params=pltpu.CompilerParams(use_tc_tiling_on_sc=True),
    )(_gather_body)


_TB = 16384
_NB = _NS // _TB


def _tpose_first_body(x_ref, o_ref):
    o_ref[:] = x_ref[:].T


def _tpose_acc_body(acc_ref, x_ref, o_ref):
    del acc_ref
    o_ref[:] = x_ref[:].T


def _transpose_slab(acc, slab, k):
    out_sd = jax.ShapeDtypeStruct((_D, _N), jnp.float32)
    out_spec = pl.BlockSpec((_D, _TB), lambda i, _k=k: (0, _k * _NB + i))
    slab_spec = pl.BlockSpec((_TB, _D), lambda i: (i, 0))
    if acc is None:
        return pl.pallas_call(
            _tpose_first_body,
            grid=(_NB,),
            in_specs=[slab_spec],
            out_specs=out_spec,
            out_shape=out_sd,
        )(slab)
    return pl.pallas_call(
        _tpose_acc_body,
        grid=(_NB,),
        in_specs=[pl.BlockSpec(memory_space=pl.ANY), slab_spec],
        out_specs=out_spec,
        out_shape=out_sd,
        input_output_aliases={0: 0},
    )(acc, slab)


def kernel(derivation_depths, inference_types, parent_counts, depth_pe,
           embed_table, Wp, bp, Wo, bo, gamma, beta):
    table = _build_table(depth_pe, embed_table, Wp, bp, Wo, bo, gamma, beta)
    gather = _make_gather()
    acc = None
    for k in range(_NSLAB):
        s = slice(k * _NS, (k + 1) * _NS)
        slab = gather(derivation_depths[s], inference_types[s],
                      parent_counts[s], table)
        acc = _transpose_slab(acc, slab, k)
    return acc.T


# speedup vs baseline: 1.0150x; 1.0013x over previous
import functools

import jax
import jax.numpy as jnp
from jax import lax
from jax.experimental import pallas as pl
from jax.experimental.pallas import tpu as pltpu
from jax.experimental.pallas import tpu_sc as plsc

_N = 1048576
_D = 64
_PD = 128
_ND = 101
_NT = 22
_NP = 8
_TBL = _ND * _NT * _NP

_NW = 32
_NSLAB = 4
_NS = _N // _NSLAB
_RPW = _NS // _NW
_C = 256
_NCHUNK = _RPW // _C


def _table_body(pe_ref, emb_ref, wp_ref, bp_ref, wo_ref, bo_ref, g_ref, b_ref,
                out_ref):
    wo = wo_ref[:]
    a = jnp.dot(pe_ref[:], wo[0:32, :], preferred_element_type=jnp.float32)
    b = jnp.dot(emb_ref[:], wo[32:48, :], preferred_element_type=jnp.float32)
    wp_o = jnp.dot(wp_ref[:], wo[48:64, :], preferred_element_type=jnp.float32)
    base = (jnp.dot(bp_ref[:], wo[48:64, :], preferred_element_type=jnp.float32)
            + bo_ref[:])
    pvals = lax.broadcasted_iota(jnp.int32, (_NP, 1), 0).astype(jnp.float32)
    c = pvals * wp_o + base
    x = (a[:, None, None, :] + b[None, :, None, :] + c[None, None, :, :])
    mean = jnp.sum(x, axis=-1, keepdims=True) * (1.0 / _D)
    var = jnp.sum(x * x, axis=-1, keepdims=True) * (1.0 / _D) - mean * mean
    out_ref[:] = (x - mean) * lax.rsqrt(var + 1e-5) * g_ref[:] + b_ref[:]


def _build_table(depth_pe, embed_table, Wp, bp, Wo, bo, gamma, beta):
    pad = _PD - _D
    table4 = pl.pallas_call(
        _table_body,
        out_shape=jax.ShapeDtypeStruct((_ND, _NT, _NP, _PD), jnp.float32),
    )(depth_pe, embed_table, Wp, bp.reshape(1, -1),
      jnp.pad(Wo, ((0, 0), (0, pad))), jnp.pad(bo, (0, pad)).reshape(1, -1),
      jnp.pad(gamma, (0, pad)).reshape(1, -1),
      jnp.pad(beta, (0, pad)).reshape(1, -1))
    return table4.reshape(_TBL, _PD)


def _gather_body(d_hbm, t_hbm, p_hbm, table_hbm, out_hbm,
                 d_v, t_v, p_v, idx0_v, idx1_v, rows0_v, rows1_v, packt_v,
                 sem):
    idx_s = (idx0_v, idx1_v)
    rows_s = (rows0_v, rows1_v)
    wid = lax.axis_index("s") * 2 + lax.axis_index("c")
    base = wid * _RPW
    last = _NCHUNK - 1

    def load_idx(slot, ci):
        off = base + ci * _C
        pltpu.sync_copy(d_hbm.at[pl.ds(off, _C)], d_v)
        pltpu.sync_copy(t_hbm.at[pl.ds(off, _C)], t_v)
        pltpu.sync_copy(p_hbm.at[pl.ds(off, _C)], p_v)
        for i in range(_C // 16):
            s = pl.ds(i * 16, 16)
            d = jnp.clip(d_v[s], 0, _ND - 1)
            t = jnp.clip(t_v[s], 0, _NT - 1)
            idx_s[slot][s] = d * (_NT * _NP) + t * _NP + p_v[s]

    def start_gather(slot):
        return pltpu.async_copy(table_hbm.at[idx_s[slot]],
                                rows_s[slot], sem)

    def wait_gather(slot):
        pltpu.make_async_copy(table_hbm.at[idx_s[slot]],
                              rows_s[slot], sem).wait()

    load_idx(0, 0)
    start_gather(0)
    load_idx(1, 1)

    def outer(gi, carry):
        for b in (0, 1):
            g = 2 * gi + b
            nb = 1 - b
            wait_gather(b)
            start_gather(nb)

            def pack_row(r, carry, _rows=rows_s[b]):
                for k in range(_D // 16):
                    s = pl.ds(k * 16, 16)
                    packt_v[r, s] = _rows[r, s]
                return carry

            lax.fori_loop(0, _C, pack_row, 0)
            pltpu.sync_copy(packt_v, out_hbm.at[pl.ds(base + g * _C, _C)])
            load_idx(b, jnp.minimum(g + 2, last))
        return carry

    lax.fori_loop(0, _NCHUNK // 2, outer, 0)
    wait_gather(0)


@functools.cache
def _make_gather():
    return functools.partial(
        pl.kernel,
        out_type=jax.ShapeDtypeStruct((_NS, _D), jnp.float32),
        mesh=plsc.VectorSubcoreMesh(core_axis_name="c", subcore_axis_name="s",
                                    num_cores=2, num_subcores=16),
        scratch_types=[
            pltpu.VMEM((_C,), jnp.int32),
            pltpu.VMEM((_C,), jnp.int32),
            pltpu.VMEM((_C,), jnp.int32),
            pltpu.VMEM((_C,), jnp.int32),
            pltpu.VMEM((_C,), jnp.int32),
            pltpu.VMEM((_C, _PD), jnp.float32),
            pltpu.VMEM((_C, _PD), jnp.float32),
            pltpu.VMEM((_C, _D), jnp.float32),
            pltpu.SemaphoreType.DMA,
        ],
        compiler_params=pltpu.CompilerParams(use_tc_tiling_on_sc=True),
    )(_gather_body)


_TB = 8192
_NB = _NS // _TB


def _tpose_first_body(x_ref, o_ref):
    o_ref[:] = x_ref[:].T


def _tpose_acc_body(acc_ref, x_ref, o_ref):
    del acc_ref
    o_ref[:] = x_ref[:].T


def _transpose_slab(acc, slab, k):
    out_sd = jax.ShapeDtypeStruct((_D, _N), jnp.float32)
    out_spec = pl.BlockSpec((_D, _TB), lambda i, _k=k: (0, _k * _NB + i))
    slab_spec = pl.BlockSpec((_TB, _D), lambda i: (i, 0))
    if acc is None:
        return pl.pallas_call(
            _tpose_first_body,
            grid=(_NB,),
            in_specs=[slab_spec],
            out_specs=out_spec,
            out_shape=out_sd,
        )(slab)
    return pl.pallas_call(
        _tpose_acc_body,
        grid=(_NB,),
        in_specs=[pl.BlockSpec(memory_space=pl.ANY), slab_spec],
        out_specs=out_spec,
        out_shape=out_sd,
        input_output_aliases={0: 0},
    )(acc, slab)


def kernel(derivation_depths, inference_types, parent_counts, depth_pe,
           embed_table, Wp, bp, Wo, bo, gamma, beta):
    table = _build_table(depth_pe, embed_table, Wp, bp, Wo, bo, gamma, beta)
    gather = _make_gather()
    acc = None
    for k in range(_NSLAB):
        s = slice(k * _NS, (k + 1) * _NS)
        slab = gather(derivation_depths[s], inference_types[s],
                      parent_counts[s], table)
        acc = _transpose_slab(acc, slab, k)
    return acc.T


# speedup vs baseline: 1.0377x; 1.0224x over previous
import functools

import jax
import jax.numpy as jnp
from jax import lax
from jax.experimental import pallas as pl
from jax.experimental.pallas import tpu as pltpu
from jax.experimental.pallas import tpu_sc as plsc

_N = 1048576
_D = 64
_PD = 128
_ND = 101
_NT = 22
_NP = 8
_TBL = _ND * _NT * _NP

_NW = 32
_NSLAB = 4
_NS = _N // _NSLAB
_RPW = _NS // _NW
_C = 256
_NCHUNK = _RPW // _C


def _table_body(pe_ref, emb_ref, wp_ref, bp_ref, wo_ref, bo_ref, g_ref, b_ref,
                out_ref):
    wo = wo_ref[:]
    a = jnp.dot(pe_ref[:], wo[0:32, :], preferred_element_type=jnp.float32)
    b = jnp.dot(emb_ref[:], wo[32:48, :], preferred_element_type=jnp.float32)
    wp_o = jnp.dot(wp_ref[:], wo[48:64, :], preferred_element_type=jnp.float32)
    base = (jnp.dot(bp_ref[:], wo[48:64, :], preferred_element_type=jnp.float32)
            + bo_ref[:])
    pvals = lax.broadcasted_iota(jnp.int32, (_NP, 1), 0).astype(jnp.float32)
    c = pvals * wp_o + base
    x = (a[:, None, None, :] + b[None, :, None, :] + c[None, None, :, :])
    mean = jnp.sum(x, axis=-1, keepdims=True) * (1.0 / _D)
    var = jnp.sum(x * x, axis=-1, keepdims=True) * (1.0 / _D) - mean * mean
    out_ref[:] = (x - mean) * lax.rsqrt(var + 1e-5) * g_ref[:] + b_ref[:]


def _build_table(depth_pe, embed_table, Wp, bp, Wo, bo, gamma, beta):
    pad = _PD - _D
    table4 = pl.pallas_call(
        _table_body,
        out_shape=jax.ShapeDtypeStruct((_ND, _NT, _NP, _PD), jnp.float32),
    )(depth_pe, embed_table, Wp, bp.reshape(1, -1),
      jnp.pad(Wo, ((0, 0), (0, pad))), jnp.pad(bo, (0, pad)).reshape(1, -1),
      jnp.pad(gamma, (0, pad)).reshape(1, -1),
      jnp.pad(beta, (0, pad)).reshape(1, -1))
    return table4.reshape(_TBL, _PD)


def _gather_body(slab_base, d_hbm, t_hbm, p_hbm, table_hbm, out_hbm,
                 d_v, t_v, p_v, idx0_v, idx1_v, rows0_v, rows1_v, packt_v,
                 sem):
    idx_s = (idx0_v, idx1_v)
    rows_s = (rows0_v, rows1_v)
    wid = lax.axis_index("s") * 2 + lax.axis_index("c")
    base = wid * _RPW
    last = _NCHUNK - 1

    def load_idx(slot, ci):
        off = slab_base + base + ci * _C
        pltpu.sync_copy(d_hbm.at[pl.ds(off, _C)], d_v)
        pltpu.sync_copy(t_hbm.at[pl.ds(off, _C)], t_v)
        pltpu.sync_copy(p_hbm.at[pl.ds(off, _C)], p_v)
        for i in range(_C // 16):
            s = pl.ds(i * 16, 16)
            d = jnp.clip(d_v[s], 0, _ND - 1)
            t = jnp.clip(t_v[s], 0, _NT - 1)
            idx_s[slot][s] = d * (_NT * _NP) + t * _NP + p_v[s]

    def start_gather(slot):
        return pltpu.async_copy(table_hbm.at[idx_s[slot]],
                                rows_s[slot], sem)

    def wait_gather(slot):
        pltpu.make_async_copy(table_hbm.at[idx_s[slot]],
                              rows_s[slot], sem).wait()

    load_idx(0, 0)
    start_gather(0)
    load_idx(1, 1)

    def outer(gi, carry):
        for b in (0, 1):
            g = 2 * gi + b
            nb = 1 - b
            wait_gather(b)
            start_gather(nb)

            def pack_row(r, carry, _rows=rows_s[b]):
                for k in range(_D // 16):
                    s = pl.ds(k * 16, 16)
                    packt_v[r, s] = _rows[r, s]
                return carry

            lax.fori_loop(0, _C, pack_row, 0)
            pltpu.sync_copy(packt_v, out_hbm.at[pl.ds(base + g * _C, _C)])
            load_idx(b, jnp.minimum(g + 2, last))
        return carry

    lax.fori_loop(0, _NCHUNK // 2, outer, 0)
    wait_gather(0)


@functools.cache
def _make_gather(slab_base):
    return functools.partial(
        pl.kernel,
        out_type=jax.ShapeDtypeStruct((_NS, _D), jnp.float32),
        mesh=plsc.VectorSubcoreMesh(core_axis_name="c", subcore_axis_name="s",
                                    num_cores=2, num_subcores=16),
        scratch_types=[
            pltpu.VMEM((_C,), jnp.int32),
            pltpu.VMEM((_C,), jnp.int32),
            pltpu.VMEM((_C,), jnp.int32),
            pltpu.VMEM((_C,), jnp.int32),
            pltpu.VMEM((_C,), jnp.int32),
            pltpu.VMEM((_C, _PD), jnp.float32),
            pltpu.VMEM((_C, _PD), jnp.float32),
            pltpu.VMEM((_C, _D), jnp.float32),
            pltpu.SemaphoreType.DMA,
        ],
        compiler_params=pltpu.CompilerParams(use_tc_tiling_on_sc=True),
    )(functools.partial(_gather_body, slab_base))


_TB = 8192
_NB = _NS // _TB


def _tpose_first_body(x_ref, o_ref):
    o_ref[:] = x_ref[:].T


def _tpose_acc_body(acc_ref, x_ref, o_ref):
    del acc_ref
    o_ref[:] = x_ref[:].T


def _transpose_slab(acc, slab, k):
    out_sd = jax.ShapeDtypeStruct((_D, _N), jnp.float32)
    out_spec = pl.BlockSpec((_D, _TB), lambda i, _k=k: (0, _k * _NB + i))
    slab_spec = pl.BlockSpec((_TB, _D), lambda i: (i, 0))
    if acc is None:
        return pl.pallas_call(
            _tpose_first_body,
            grid=(_NB,),
            in_specs=[slab_spec],
            out_specs=out_spec,
            out_shape=out_sd,
        )(slab)
    return pl.pallas_call(
        _tpose_acc_body,
        grid=(_NB,),
        in_specs=[pl.BlockSpec(memory_space=pl.ANY), slab_spec],
        out_specs=out_spec,
        out_shape=out_sd,
        input_output_aliases={0: 0},
    )(acc, slab)


def kernel(derivation_depths, inference_types, parent_counts, depth_pe,
           embed_table, Wp, bp, Wo, bo, gamma, beta):
    table = _build_table(depth_pe, embed_table, Wp, bp, Wo, bo, gamma, beta)
    acc = None
    for k in range(_NSLAB):
        slab = _make_gather(k * _NS)(derivation_depths, inference_types,
                                     parent_counts, table)
        acc = _transpose_slab(acc, slab, k)
    return acc.T
